# baseline (speedup 1.0000x reference)
import functools

import jax
import jax.numpy as jnp
from jax.experimental import pallas as pl

T_GRAPHS = 112
BATCH = 16
SEQ_LEN = 7
H_DIM = 512
DOCS_PER_GRAPH = 20


def _linear_kernel(x_ref, w_ref, b_ref, o_ref, *, input_relu):
    x = x_ref[...]
    if input_relu:
        x = jnp.maximum(x, 0.0)
    o_ref[...] = (
        jnp.dot(x, w_ref[...], preferred_element_type=jnp.float32) + b_ref[...]
    )


def _linear(x, W, b, input_relu=False, block_m=2048):
    n, k = x.shape
    h = W.shape[0]
    wt = W.T
    n_pad = ((n + block_m - 1) // block_m) * block_m
    if n_pad != n:
        x = jnp.pad(x, ((0, n_pad - n), (0, 0)))
    out = pl.pallas_call(
        functools.partial(_linear_kernel, input_relu=input_relu),
        grid=(n_pad // block_m,),
        in_specs=[
            pl.BlockSpec((block_m, k), lambda i: (i, 0)),
            pl.BlockSpec((k, h), lambda i: (0, 0)),
            pl.BlockSpec((1, h), lambda i: (0, 0)),
        ],
        out_specs=pl.BlockSpec((block_m, h), lambda i: (i, 0)),
        out_shape=jax.ShapeDtypeStruct((n_pad, h), jnp.float32),
    )(x, wt, b.reshape(1, h))
    return out[:n]


def _tail_kernel(hd_ref, wih0_ref, whh0_ref, bih0_ref, bhh0_ref,
                 wih1_ref, whh1_ref, bih1_ref, bhh1_ref,
                 ow_ref, ob_ref, y_ref, loss_ref, sig_ref):
    hd = hd_ref[...]
    pooled = jnp.max(hd.reshape(T_GRAPHS, DOCS_PER_GRAPH, H_DIM), axis=1)
    pooled = jnp.maximum(pooled, 0.0)
    x = pooled.reshape(BATCH, SEQ_LEN, H_DIM)

    wih0 = wih0_ref[...]
    whh0 = whh0_ref[...]
    bih0 = bih0_ref[...]
    bhh0 = bhh0_ref[...]
    wih1 = wih1_ref[...]
    whh1 = whh1_ref[...]
    bih1 = bih1_ref[...]
    bhh1 = bhh1_ref[...]

    h0 = jnp.zeros((BATCH, H_DIM), jnp.float32)
    ys = []
    for t in range(SEQ_LEN):
        xt = x[:, t, :]
        h0 = jnp.tanh(
            jnp.dot(xt, wih0, preferred_element_type=jnp.float32) + bih0
            + jnp.dot(h0, whh0, preferred_element_type=jnp.float32) + bhh0
        )
        ys.append(h0)
    h1 = jnp.zeros((BATCH, H_DIM), jnp.float32)
    for t in range(SEQ_LEN):
        h1 = jnp.tanh(
            jnp.dot(ys[t], wih1, preferred_element_type=jnp.float32) + bih1
            + jnp.dot(h1, whh1, preferred_element_type=jnp.float32) + bhh1
        )
    logits = jnp.dot(h1, ow_ref[...], preferred_element_type=jnp.float32) + ob_ref[...]
    l = logits[:, 0]
    y = y_ref[0, :]
    loss = jnp.mean(
        jnp.maximum(l, 0.0) - l * y + jnp.log1p(jnp.exp(-jnp.abs(l)))
    )
    loss_ref[...] = loss.reshape(1, 1)
    sig_ref[...] = jax.nn.sigmoid(logits)


def _tail(hd, rnn_Wih, rnn_Whh, rnn_bih, rnn_bhh, out_W, out_b, y_data):
    loss, sig = pl.pallas_call(
        _tail_kernel,
        out_shape=[
            jax.ShapeDtypeStruct((1, 1), jnp.float32),
            jax.ShapeDtypeStruct((BATCH, 1), jnp.float32),
        ],
    )(
        hd,
        rnn_Wih[0].T, rnn_Whh[0].T, rnn_bih[0].reshape(1, -1), rnn_bhh[0].reshape(1, -1),
        rnn_Wih[1].T, rnn_Whh[1].T, rnn_bih[1].reshape(1, -1), rnn_bhh[1].reshape(1, -1),
        out_W.T, out_b.reshape(1, -1), y_data.reshape(1, -1),
    )
    return loss[0, 0], sig[:, 0]


def _compose(Ws, bs):
    W = Ws[0]
    b = bs[0]
    for Wi, bi in zip(Ws[1:], bs[1:]):
        W = Wi @ W
        b = b @ Wi.T + bi
    return W, b


def _mean_agg(rows, w, dst, n):
    s = jax.ops.segment_sum(rows * w[:, None], dst, num_segments=n)
    c = jax.ops.segment_sum(jnp.ones((rows.shape[0], 1), rows.dtype), dst,
                            num_segments=n)
    return s / jnp.maximum(c, 1.0)


def kernel(word_ids, topic_ids, doc_graph_ids, ww_src, ww_dst, wt_src, wt_dst,
           wd_src, wd_dst, td_src, td_dst, tt_src, tt_dst,
           ww_w, wt_w, wd_w, td_w, tt_w,
           word_embeds, topic_embeds, adapt_W, adapt_b, conv_W, conv_b,
           rnn_Wih, rnn_Whh, rnn_bih, rnn_bhh, out_W, out_b, y_data):
    n_word = word_ids.shape[0]
    n_topic = topic_ids.shape[0]
    n_doc = doc_graph_ids.shape[0]

    W1w, b1w = _compose(
        [adapt_W, conv_W[0, 0], conv_W[0, 1], conv_W[0, 2]],
        [adapt_b, conv_b[0, 0], conv_b[0, 1], conv_b[0, 2]],
    )
    W1t, b1t = _compose([conv_W[0, 3], conv_W[0, 4]],
                        [conv_b[0, 3], conv_b[0, 4]])
    W2w, b2w = _compose([conv_W[1, 0], conv_W[1, 1], conv_W[1, 2]],
                        [conv_b[1, 0], conv_b[1, 1], conv_b[1, 2]])
    W2t, b2t = _compose([conv_W[1, 3], conv_W[1, 4]],
                        [conv_b[1, 3], conv_b[1, 4]])

    hw1 = _linear(word_embeds[word_ids], W1w, b1w)
    ht1 = _linear(topic_embeds[topic_ids], W1t, b1t)

    new_w = _mean_agg(hw1[ww_src], ww_w, ww_dst, n_word)
    new_t = (_mean_agg(hw1[wt_src], wt_w, wt_dst, n_topic)
             + _mean_agg(ht1[tt_src], tt_w, tt_dst, n_topic))

    hw2 = _linear(new_w, W2w, b2w, input_relu=True)
    ht2 = _linear(new_t, W2t, b2t, input_relu=True)

    hd = (_mean_agg(hw2[wd_src], wd_w, wd_dst, n_doc)
          + _mean_agg(ht2[td_src], td_w, td_dst, n_doc))

    return _tail(hd, rnn_Wih, rnn_Whh, rnn_bih, rnn_bhh, out_W, out_b, y_data)
